# baseline (speedup 1.0000x reference)
import functools

import jax
import jax.numpy as jnp
from jax import lax
from jax.experimental import pallas as pl
from jax.experimental.pallas import tpu as pltpu
from jax.experimental.pallas import tpu_sc as plsc

NC = 2
NS = 16
NW = NC * NS
LB = 128
RB = 1024


def _mesh():
    return plsc.VectorSubcoreMesh(core_axis_name="c", subcore_axis_name="s")


def _make_hist(acc_len, n_batch):
    wpt = acc_len // NS

    @functools.partial(
        pl.kernel,
        out_type=jax.ShapeDtypeStruct((NC * acc_len,), jnp.float32),
        mesh=_mesh(),
        scratch_types=[
            pltpu.VMEM((n_batch, LB), jnp.int32),
            pltpu.VMEM((LB,), jnp.float32),
            pltpu.VMEM_SHARED((acc_len,), jnp.float32),
        ],
    )
    def hist(idx_hbm, zeros_hbm, out_hbm, idx_v, ones_v, acc):
        cid = lax.axis_index("c")
        sid = lax.axis_index("s")
        wid = sid * NC + cid
        base = sid * wpt
        pltpu.sync_copy(zeros_hbm.at[pl.ds(base, wpt)], acc.at[pl.ds(base, wpt)])
        pltpu.sync_copy(idx_hbm.at[wid], idx_v)
        for i in range(LB // 16):
            ones_v[pl.ds(i * 16, 16)] = jnp.ones((16,), jnp.float32)
        plsc.subcore_barrier()

        def body(j, carry):
            pltpu.sync_copy(ones_v, acc.at[idx_v.at[j]], add=True)
            return carry

        lax.fori_loop(0, n_batch, body, 0)
        plsc.subcore_barrier()
        pltpu.sync_copy(acc.at[pl.ds(base, wpt)],
                        out_hbm.at[pl.ds(cid * acc_len + base, wpt)])

    return hist


def _make_seg_sum(n_st, n_acc, d, n_batch):
    rpt = n_acc // NS
    tpt = n_st // NS

    @functools.partial(
        pl.kernel,
        out_type=jax.ShapeDtypeStruct((NC * n_acc, d), jnp.float32),
        mesh=_mesh(),
        scratch_types=[
            pltpu.VMEM((n_batch, LB), jnp.int32),
            pltpu.VMEM((n_batch, LB), jnp.int32),
            pltpu.VMEM((2, LB, d), jnp.float32),
            pltpu.VMEM_SHARED((n_acc, d), jnp.float32),
            pltpu.VMEM_SHARED((n_st, d), jnp.float32),
            pltpu.SemaphoreType.DMA,
            pltpu.SemaphoreType.DMA,
        ],
    )
    def seg(table_hbm, src_hbm, dst_hbm, zeros_hbm, out_hbm,
            idx_s, idx_d, rows, acc, ztab, sem0, sem1):
        cid = lax.axis_index("c")
        sid = lax.axis_index("s")
        wid = sid * NC + cid
        rbase = sid * rpt
        tb = sid * tpt
        pltpu.sync_copy(table_hbm.at[pl.ds(tb, tpt)], ztab.at[pl.ds(tb, tpt)])
        pltpu.sync_copy(zeros_hbm.at[pl.ds(rbase, rpt)],
                        acc.at[pl.ds(rbase, rpt)])
        pltpu.sync_copy(src_hbm.at[wid], idx_s)
        pltpu.sync_copy(dst_hbm.at[wid], idx_d)
        plsc.subcore_barrier()

        def body(g, carry):
            j0 = 2 * g
            j1 = j0 + 1
            c0 = pltpu.async_copy(ztab.at[idx_s.at[j0]], rows.at[0], sem0)
            c1 = pltpu.async_copy(ztab.at[idx_s.at[j1]], rows.at[1], sem1)
            c0.wait()
            pltpu.sync_copy(rows.at[0], acc.at[idx_d.at[j0]], add=True)
            c1.wait()
            pltpu.sync_copy(rows.at[1], acc.at[idx_d.at[j1]], add=True)
            return carry

        lax.fori_loop(0, n_batch // 2, body, 0)
        plsc.subcore_barrier()
        pltpu.sync_copy(acc.at[pl.ds(rbase, rpt)],
                        out_hbm.at[pl.ds(cid * n_acc + rbase, rpt)])

    return seg


def _norm_from(deg_ref):
    deg = jnp.sum(deg_ref[...], axis=1, keepdims=True)
    return lax.rsqrt(jnp.maximum(deg, 1.0))


def _tc1_body(deg_ref, lg_ref, ft_ref, wp_ref, bp_ref, w1a_ref, w1b_ref,
              oa_ref, ob_ref):
    no = _norm_from(deg_ref)
    feat = jnp.dot(ft_ref[...], wp_ref[...],
                   preferred_element_type=jnp.float32) + bp_ref[...]
    a = jnp.dot(lg_ref[...] * no, w1a_ref[...],
                preferred_element_type=jnp.float32)
    b = jnp.dot(feat * no, w1b_ref[...], preferred_element_type=jnp.float32)
    z = a + b
    h = z.shape[1] // 2
    oa_ref[...] = z[:, :h]
    ob_ref[...] = z[:, h:]


def _tc2_body(aa_ref, ab_ref, degi_ref, dego_ref, b1_ref, w2a_ref, w2b_ref,
              o_ref):
    ni = _norm_from(degi_ref)
    no = _norm_from(dego_ref)
    h = b1_ref.shape[1] // 2
    xa = jnp.maximum((aa_ref[0] + aa_ref[1]) * ni + b1_ref[:, :h], 0.0) * no
    xb = jnp.maximum((ab_ref[0] + ab_ref[1]) * ni + b1_ref[:, h:], 0.0) * no
    o_ref[...] = (
        jnp.dot(xa, w2a_ref[...], preferred_element_type=jnp.float32)
        + jnp.dot(xb, w2b_ref[...], preferred_element_type=jnp.float32))


def _tc3_body(agg_ref, degi_ref, b2_ref, o_ref):
    ni = _norm_from(degi_ref)
    o_ref[...] = (agg_ref[0] + agg_ref[1]) * ni + b2_ref[...]


def kernel(logits, features, edge_index, W_proj, b_proj, W1, b1, W2, b2):
    n, n_cls = logits.shape
    fdim = features.shape[1]
    fh = W_proj.shape[1]
    hid = W1.shape[1]
    out_dim = W2.shape[1]
    e = edge_index.shape[1]

    grid = -(-n // RB)
    n_acc = grid * RB
    n_st = -(-n // (NS * 8)) * (NS * 8)
    hh = hid // 2
    d2 = -(-out_dim // 16) * 16

    src = edge_index[0]
    dst = edge_index[1]

    cap = NW * LB
    nb1 = -(-e // cap)
    nb1 += nb1 % 2
    pad1 = nb1 * cap - e
    src3 = jnp.concatenate(
        [src, jnp.zeros((pad1,), jnp.int32)]).reshape(NW, nb1, LB)
    dst3 = jnp.concatenate(
        [dst, jnp.full((pad1,), n, jnp.int32)]).reshape(NW, nb1, LB)

    acc_len = 2 * n_acc
    nb2 = -(-2 * e // cap)
    nb2 += nb2 % 2
    pad2 = nb2 * cap - 2 * e
    both = jnp.concatenate([src, dst + n_acc, jnp.full((pad2,), n, jnp.int32)])
    idx2 = both.reshape(NW, nb2, LB)

    zeros_flat = jnp.zeros((acc_len,), jnp.float32)
    hp0 = jnp.zeros((acc_len,), jnp.float32).at[both].add(1.0)
    hp = jnp.stack([hp0, jnp.zeros((acc_len,), jnp.float32)])
    deg_out_p = hp[:, :n_acc].T
    deg_in_p = hp[:, n_acc:].T

    w1a = W1[:n_cls]
    w1b = W1[n_cls:]
    z1a, z1b = pl.pallas_call(
        _tc1_body,
        grid=(grid,),
        in_specs=[
            pl.BlockSpec((RB, NC), lambda i: (i, 0)),
            pl.BlockSpec((RB, n_cls), lambda i: (i, 0)),
            pl.BlockSpec((RB, fdim), lambda i: (i, 0)),
            pl.BlockSpec((fdim, fh), lambda i: (0, 0)),
            pl.BlockSpec((1, fh), lambda i: (0, 0)),
            pl.BlockSpec((n_cls, hid), lambda i: (0, 0)),
            pl.BlockSpec((fh, hid), lambda i: (0, 0)),
        ],
        out_specs=[
            pl.BlockSpec((RB, hh), lambda i: (i, 0)),
            pl.BlockSpec((RB, hh), lambda i: (i, 0)),
        ],
        out_shape=[
            jax.ShapeDtypeStruct((n_acc, hh), jnp.float32),
            jax.ShapeDtypeStruct((n_acc, hh), jnp.float32),
        ],
    )(deg_out_p, logits, features, W_proj, b_proj.reshape(1, fh), w1a, w1b)

    zeros_h = jnp.zeros((n_acc, hh), jnp.float32)
    sf = src3.reshape(-1); df = dst3.reshape(-1)
    agg1a = jnp.stack([jnp.zeros((n_acc, hh), jnp.float32).at[df].add(z1a[sf]),
                       jnp.zeros((n_acc, hh), jnp.float32)])
    agg1b = jnp.stack([jnp.zeros((n_acc, hh), jnp.float32).at[df].add(z1b[sf]),
                       jnp.zeros((n_acc, hh), jnp.float32)])

    w2p = jnp.pad(W2, ((0, 0), (0, d2 - out_dim)))
    b1r = b1.reshape(1, hid)
    z2 = pl.pallas_call(
        _tc2_body,
        grid=(grid,),
        in_specs=[
            pl.BlockSpec((NC, RB, hh), lambda i: (0, i, 0)),
            pl.BlockSpec((NC, RB, hh), lambda i: (0, i, 0)),
            pl.BlockSpec((RB, NC), lambda i: (i, 0)),
            pl.BlockSpec((RB, NC), lambda i: (i, 0)),
            pl.BlockSpec((1, hid), lambda i: (0, 0)),
            pl.BlockSpec((hh, d2), lambda i: (0, 0)),
            pl.BlockSpec((hh, d2), lambda i: (0, 0)),
        ],
        out_specs=pl.BlockSpec((RB, d2), lambda i: (i, 0)),
        out_shape=jax.ShapeDtypeStruct((n_acc, d2), jnp.float32),
    )(agg1a, agg1b, deg_in_p, deg_out_p, b1r, w2p[:hh], w2p[hh:])

    zeros2 = jnp.zeros((n_acc, d2), jnp.float32)
    agg2 = jnp.stack([jnp.zeros((n_acc, d2), jnp.float32).at[df].add(z2[sf]),
                      jnp.zeros((n_acc, d2), jnp.float32)])

    b2r = jnp.pad(b2, (0, d2 - out_dim)).reshape(1, d2)
    out48 = pl.pallas_call(
        _tc3_body,
        grid=(grid,),
        in_specs=[
            pl.BlockSpec((NC, RB, d2), lambda i: (0, i, 0)),
            pl.BlockSpec((RB, NC), lambda i: (i, 0)),
            pl.BlockSpec((1, d2), lambda i: (0, 0)),
        ],
        out_specs=pl.BlockSpec((RB, d2), lambda i: (i, 0)),
        out_shape=jax.ShapeDtypeStruct((n_acc, d2), jnp.float32),
    )(agg2, deg_in_p, b2r)

    return out48[:n, :out_dim]


# speedup vs baseline: 4.7155x; 4.7155x over previous
import functools

import jax
import jax.numpy as jnp
from jax import lax
from jax.experimental import pallas as pl
from jax.experimental.pallas import tpu as pltpu
from jax.experimental.pallas import tpu_sc as plsc

NC = 2
NS = 16
NW = NC * NS
LB = 128
RB = 1024
D = 128
TR = 128


def _mesh():
    return plsc.VectorSubcoreMesh(core_axis_name="c", subcore_axis_name="s")


def _chunks(total):
    out = []
    off = 0
    while off < total:
        cs = min(LB, total - off)
        out.append((off, cs))
        off += cs
    return out


def _make_hist(acc_len, n_batch):
    wpt = acc_len // NS

    @functools.partial(
        pl.kernel,
        out_type=jax.ShapeDtypeStruct((NC * acc_len,), jnp.float32),
        mesh=_mesh(),
        scratch_types=[
            pltpu.VMEM((n_batch, LB), jnp.int32),
            pltpu.VMEM((LB,), jnp.float32),
            pltpu.VMEM_SHARED((acc_len,), jnp.float32),
        ],
    )
    def hist(idx_hbm, zeros_hbm, out_hbm, idx_v, ones_v, acc):
        cid = lax.axis_index("c")
        sid = lax.axis_index("s")
        wid = sid * NC + cid
        base = sid * wpt
        pltpu.sync_copy(zeros_hbm.at[pl.ds(base, wpt)], acc.at[pl.ds(base, wpt)])
        pltpu.sync_copy(idx_hbm.at[wid], idx_v)
        for i in range(LB // 16):
            ones_v[pl.ds(i * 16, 16)] = jnp.ones((16,), jnp.float32)
        plsc.subcore_barrier()

        def body(j, carry):
            pltpu.sync_copy(ones_v, acc.at[idx_v.at[j]], add=True)
            return carry

        lax.fori_loop(0, n_batch, body, 0)
        plsc.subcore_barrier()
        pltpu.sync_copy(acc.at[pl.ds(base, wpt)],
                        out_hbm.at[pl.ds(cid * acc_len + base, wpt)])

    return hist


def _make_seg_sum(n_acc, n_batch):
    half = n_acc // NC
    zpt = (half + TR) // NS
    wpt = half // NS

    @functools.partial(
        pl.kernel,
        out_type=jax.ShapeDtypeStruct((n_acc, D), jnp.float32),
        mesh=_mesh(),
        scratch_types=[
            pltpu.VMEM((n_batch, LB), jnp.int32),
            pltpu.VMEM((n_batch, LB), jnp.int32),
            pltpu.VMEM((LB, D), jnp.float32),
            pltpu.VMEM((LB, D), jnp.float32),
            pltpu.VMEM_SHARED((half + TR, D), jnp.float32),
            pltpu.SemaphoreType.DMA,
            pltpu.SemaphoreType.DMA,
        ],
    )
    def seg(table_hbm, src_hbm, dst_hbm, zeros_hbm, out_hbm,
            idx_s, idx_d, r0, r1, acc, sem0, sem1):
        cid = lax.axis_index("c")
        sid = lax.axis_index("s")
        pltpu.sync_copy(zeros_hbm, r0)
        for off, cs in _chunks(zpt):
            pltpu.sync_copy(r0.at[pl.ds(0, cs)],
                            acc.at[pl.ds(sid * zpt + off, cs)])
        pltpu.sync_copy(src_hbm.at[sid], idx_s)
        pltpu.sync_copy(dst_hbm.at[cid, sid], idx_d)
        plsc.subcore_barrier()

        def body(g, carry):
            j0 = 2 * g
            j1 = j0 + 1
            c0 = pltpu.async_copy(table_hbm.at[idx_s.at[j0]], r0, sem0)
            c1 = pltpu.async_copy(table_hbm.at[idx_s.at[j1]], r1, sem1)
            c0.wait()
            pltpu.sync_copy(r0, acc.at[idx_d.at[j0]], add=True)
            c1.wait()
            pltpu.sync_copy(r1, acc.at[idx_d.at[j1]], add=True)
            return carry

        lax.fori_loop(0, n_batch // 2, body, 0)
        plsc.subcore_barrier()
        for off, cs in _chunks(wpt):
            pltpu.sync_copy(acc.at[pl.ds(TR + sid * wpt + off, cs)],
                            r0.at[pl.ds(0, cs)])
            pltpu.sync_copy(r0.at[pl.ds(0, cs)],
                            out_hbm.at[pl.ds(cid * half + sid * wpt + off, cs)])

    return seg


def _norm_from(deg_ref):
    deg = jnp.sum(deg_ref[...], axis=1, keepdims=True)
    return lax.rsqrt(jnp.maximum(deg, 1.0))


def _tc1_body(deg_ref, lg_ref, ft_ref, wp_ref, bp_ref, w1a_ref, w1b_ref,
              o_ref):
    no = _norm_from(deg_ref)
    feat = jnp.dot(ft_ref[...], wp_ref[...],
                   preferred_element_type=jnp.float32) + bp_ref[...]
    a = jnp.dot(lg_ref[...] * no, w1a_ref[...],
                preferred_element_type=jnp.float32)
    b = jnp.dot(feat * no, w1b_ref[...], preferred_element_type=jnp.float32)
    o_ref[...] = a + b


def _tc2_body(agg_ref, degi_ref, dego_ref, b1_ref, w2_ref, o_ref):
    ni = _norm_from(degi_ref)
    no = _norm_from(dego_ref)
    h = w2_ref.shape[0]
    x = agg_ref[:, :h] * ni + b1_ref[...]
    x = jnp.maximum(x, 0.0) * no
    o_ref[...] = jnp.dot(x, w2_ref[...], preferred_element_type=jnp.float32)


def _tc3_body(agg_ref, degi_ref, b2_ref, o_ref):
    ni = _norm_from(degi_ref)
    h = b2_ref.shape[1]
    o_ref[...] = agg_ref[:, :h] * ni + b2_ref[...]


def kernel(logits, features, edge_index, W_proj, b_proj, W1, b1, W2, b2):
    n, n_cls = logits.shape
    fdim = features.shape[1]
    fh = W_proj.shape[1]
    hid = W1.shape[1]
    out_dim = W2.shape[1]
    e = edge_index.shape[1]

    grid = -(-n // RB)
    n_acc = grid * RB
    half = n_acc // NC

    src = edge_index[0]
    dst = edge_index[1]

    capc = NS * LB
    nbc = -(-e // capc)
    nbc += nbc % 2
    padc = nbc * capc - e
    src2 = jnp.concatenate(
        [src, jnp.zeros((padc,), jnp.int32)]).reshape(NS, nbc, LB)
    dloc = []
    for c in range(NC):
        lo = c * half
        in_rng = (dst >= lo) & (dst < lo + half)
        loc = jnp.where(in_rng, dst - lo + TR, dst % TR)
        dloc.append(jnp.concatenate([loc, jnp.zeros((padc,), jnp.int32)]))
    dst2 = jnp.stack(dloc).reshape(NC, NS, nbc, LB)

    cap = NW * LB
    acc_len = 2 * n_acc
    nb2 = -(-2 * e // cap)
    nb2 += nb2 % 2
    pad2 = nb2 * cap - 2 * e
    both = jnp.concatenate([src, dst + n_acc, jnp.full((pad2,), n, jnp.int32)])
    idx2 = both.reshape(NW, nb2, LB)

    zeros_flat = jnp.zeros((acc_len,), jnp.float32)
    hist_parts = _make_hist(acc_len, nb2)(idx2, zeros_flat)
    hp = hist_parts.reshape(NC, acc_len)
    deg_out_p = hp[:, :n_acc].T
    deg_in_p = hp[:, n_acc:].T

    w1a = jnp.pad(W1[:n_cls], ((0, 0), (0, D - hid)))
    w1b = jnp.pad(W1[n_cls:], ((0, 0), (0, D - hid)))
    z1 = pl.pallas_call(
        _tc1_body,
        grid=(grid,),
        in_specs=[
            pl.BlockSpec((RB, NC), lambda i: (i, 0)),
            pl.BlockSpec((RB, n_cls), lambda i: (i, 0)),
            pl.BlockSpec((RB, fdim), lambda i: (i, 0)),
            pl.BlockSpec((fdim, fh), lambda i: (0, 0)),
            pl.BlockSpec((1, fh), lambda i: (0, 0)),
            pl.BlockSpec((n_cls, D), lambda i: (0, 0)),
            pl.BlockSpec((fh, D), lambda i: (0, 0)),
        ],
        out_specs=pl.BlockSpec((RB, D), lambda i: (i, 0)),
        out_shape=jax.ShapeDtypeStruct((n_acc, D), jnp.float32),
    )(deg_out_p, logits, features, W_proj, b_proj.reshape(1, fh), w1a, w1b)

    zeros_nd = jnp.zeros((LB, D), jnp.float32)
    seg = _make_seg_sum(n_acc, nbc)
    agg1 = seg(z1, src2, dst2, zeros_nd)

    w2p = jnp.pad(W2, ((0, 0), (0, D - out_dim)))
    b1r = b1.reshape(1, hid)
    z2 = pl.pallas_call(
        _tc2_body,
        grid=(grid,),
        in_specs=[
            pl.BlockSpec((RB, D), lambda i: (i, 0)),
            pl.BlockSpec((RB, NC), lambda i: (i, 0)),
            pl.BlockSpec((RB, NC), lambda i: (i, 0)),
            pl.BlockSpec((1, hid), lambda i: (0, 0)),
            pl.BlockSpec((hid, D), lambda i: (0, 0)),
        ],
        out_specs=pl.BlockSpec((RB, D), lambda i: (i, 0)),
        out_shape=jax.ShapeDtypeStruct((n_acc, D), jnp.float32),
    )(agg1, deg_in_p, deg_out_p, b1r, w2p)

    agg2 = seg(z2, src2, dst2, zeros_nd)

    b2r = jnp.pad(b2, (0, hid - out_dim)).reshape(1, hid)
    outp = pl.pallas_call(
        _tc3_body,
        grid=(grid,),
        in_specs=[
            pl.BlockSpec((RB, D), lambda i: (i, 0)),
            pl.BlockSpec((RB, NC), lambda i: (i, 0)),
            pl.BlockSpec((1, hid), lambda i: (0, 0)),
        ],
        out_specs=pl.BlockSpec((RB, hid), lambda i: (i, 0)),
        out_shape=jax.ShapeDtypeStruct((n_acc, hid), jnp.float32),
    )(agg2, deg_in_p, b2r)

    return outp[:n, :out_dim]
